# baseline (speedup 1.0000x reference)
import functools

import jax
import jax.numpy as jnp
from jax import lax
from jax.experimental import pallas as pl
from jax.experimental.pallas import tpu as pltpu
from jax.experimental.pallas import tpu_sc as plsc

N = 512
D = 256
NW = 32
JW = N // NW
KI = 8
NB = 2
NCH = N // KI
LANES = 16


def _sc_body(x_hbm, pe_hbm, out_hbm, pe_v, buf, lsem, ssem):
    wid = lax.axis_index("s") * 2 + lax.axis_index("c")
    jb = wid * JW
    pltpu.sync_copy(pe_hbm.at[pl.ds(jb, JW), :], pe_v)

    def x_slc(c):
        return x_hbm.at[pl.ds(c * KI, KI), pl.ds(jb, JW), :]

    def o_slc(c):
        return out_hbm.at[pl.ds(c * KI, KI), pl.ds(jb, JW), :]

    for b in range(NB):
        pltpu.async_copy(x_slc(b), buf.at[b], lsem.at[b])

    @pl.loop(0, NCH, step=NB)
    def _chunk_loop(c0):
        for b in range(NB):
            c = c0 + b
            pltpu.make_async_copy(x_slc(c), buf.at[b], lsem.at[b]).wait()

            @plsc.parallel_loop(0, JW, unroll=2)
            def _j_loop(j):
                for dv in range(D // LANES):
                    dslc = pl.ds(dv * LANES, LANES)
                    pe_reg = pe_v[j, dslc]
                    for k in range(KI):
                        buf[b, k, j, dslc] = buf[b, k, j, dslc] + pe_reg

            pltpu.async_copy(buf.at[b], o_slc(c), ssem.at[b])

        for b in range(NB):
            c = c0 + b
            pltpu.make_async_copy(buf.at[b], o_slc(c), ssem.at[b]).wait()

            @pl.when(c + NB < NCH)
            def _refill():
                pltpu.async_copy(x_slc(c + NB), buf.at[b], lsem.at[b])


@jax.jit
def kernel(x, pe_weight):
    mesh = plsc.VectorSubcoreMesh(core_axis_name="c", subcore_axis_name="s")
    run = functools.partial(
        pl.kernel,
        mesh=mesh,
        out_type=jax.ShapeDtypeStruct((N, N, D), jnp.float32),
        scratch_types=[
            pltpu.VMEM((JW, D), jnp.float32),
            pltpu.VMEM((NB, KI, JW, D), jnp.float32),
            pltpu.SemaphoreType.DMA((NB,)),
            pltpu.SemaphoreType.DMA((NB,)),
        ],
    )(_sc_body)
    return run(x, pe_weight)


# speedup vs baseline: 1.1686x; 1.1686x over previous
import functools

import jax
import jax.numpy as jnp
from jax import lax
from jax.experimental import pallas as pl
from jax.experimental.pallas import tpu as pltpu
from jax.experimental.pallas import tpu_sc as plsc

N = 512
D = 256
NW = 32
JW = N // NW
KI = 4
NB = 2
NCH = N // KI
LANES = 16


def _sc_body(x_hbm, pe_hbm, out_hbm, pe_v, bin_, bout, lsem, ssem):
    wid = lax.axis_index("s") * 2 + lax.axis_index("c")
    jb = wid * JW
    pltpu.sync_copy(pe_hbm.at[pl.ds(jb, JW), :], pe_v)

    def x_slc(c):
        return x_hbm.at[pl.ds(c * KI, KI), pl.ds(jb, JW), :]

    def o_slc(c):
        return out_hbm.at[pl.ds(c * KI, KI), pl.ds(jb, JW), :]

    for b in range(NB):
        pltpu.async_copy(x_slc(b), bin_.at[b], lsem.at[b])

    @pl.loop(0, NCH, step=NB)
    def _chunk_loop(c0):
        for b in range(NB):
            c = c0 + b
            pltpu.make_async_copy(x_slc(c), bin_.at[b], lsem.at[b]).wait()

            @pl.when(c >= NB)
            def _drain_out():
                pltpu.make_async_copy(
                    bout.at[b], o_slc(c - NB), ssem.at[b]
                ).wait()

            @plsc.parallel_loop(0, JW, unroll=2)
            def _j_loop(j):
                for dv in range(D // LANES):
                    dslc = pl.ds(dv * LANES, LANES)
                    pe_reg = pe_v[j, dslc]
                    for k in range(KI):
                        bout[b, k, j, dslc] = bin_[b, k, j, dslc] + pe_reg

            pltpu.async_copy(bout.at[b], o_slc(c), ssem.at[b])

            @pl.when(c + NB < NCH)
            def _refill():
                pltpu.async_copy(x_slc(c + NB), bin_.at[b], lsem.at[b])

    for b in range(NB):
        pltpu.make_async_copy(
            bout.at[b], o_slc(NCH - NB + b), ssem.at[b]
        ).wait()


@jax.jit
def kernel(x, pe_weight):
    mesh = plsc.VectorSubcoreMesh(core_axis_name="c", subcore_axis_name="s")
    run = functools.partial(
        pl.kernel,
        mesh=mesh,
        out_type=jax.ShapeDtypeStruct((N, N, D), jnp.float32),
        scratch_types=[
            pltpu.VMEM((JW, D), jnp.float32),
            pltpu.VMEM((NB, KI, JW, D), jnp.float32),
            pltpu.VMEM((NB, KI, JW, D), jnp.float32),
            pltpu.SemaphoreType.DMA((NB,)),
            pltpu.SemaphoreType.DMA((NB,)),
        ],
    )(_sc_body)
    return run(x, pe_weight)


# speedup vs baseline: 1.1719x; 1.0028x over previous
import functools

import jax
import jax.numpy as jnp
from jax import lax
from jax.experimental import pallas as pl
from jax.experimental.pallas import tpu as pltpu
from jax.experimental.pallas import tpu_sc as plsc

N = 512
D = 256
NW = 32
JW = N // NW
KI = 4
NB = 2
NCH = N // KI
LANES = 16


def _sc_body(x_hbm, pe_hbm, out_hbm, pe_v, bin_, bout, lsem, ssem):
    wid = lax.axis_index("s") * 2 + lax.axis_index("c")
    jb = wid * JW
    def x_slc(c):
        return x_hbm.at[pl.ds(c * KI, KI), pl.ds(jb, JW), :]

    def o_slc(c):
        return out_hbm.at[pl.ds(c * KI, KI), pl.ds(jb, JW), :]

    for b in range(NB):
        pltpu.async_copy(x_slc(b), bin_.at[b], lsem.at[b])

    pltpu.sync_copy(pe_hbm.at[pl.ds(jb, JW), :], pe_v)

    @pl.loop(0, NCH, step=NB)
    def _chunk_loop(c0):
        for b in range(NB):
            c = c0 + b
            pltpu.make_async_copy(x_slc(c), bin_.at[b], lsem.at[b]).wait()

            @pl.when(c >= NB)
            def _drain_out():
                pltpu.make_async_copy(
                    bout.at[b], o_slc(c - NB), ssem.at[b]
                ).wait()

            @plsc.parallel_loop(0, JW, unroll=2)
            def _j_loop(j):
                for dv in range(D // LANES):
                    dslc = pl.ds(dv * LANES, LANES)
                    pe_reg = pe_v[j, dslc]
                    for k in range(KI):
                        bout[b, k, j, dslc] = bin_[b, k, j, dslc] + pe_reg

            pltpu.async_copy(bout.at[b], o_slc(c), ssem.at[b])

            @pl.when(c + NB < NCH)
            def _refill():
                pltpu.async_copy(x_slc(c + NB), bin_.at[b], lsem.at[b])

    for b in range(NB):
        pltpu.make_async_copy(
            bout.at[b], o_slc(NCH - NB + b), ssem.at[b]
        ).wait()


@jax.jit
def kernel(x, pe_weight):
    mesh = plsc.VectorSubcoreMesh(core_axis_name="c", subcore_axis_name="s")
    run = functools.partial(
        pl.kernel,
        mesh=mesh,
        out_type=jax.ShapeDtypeStruct((N, N, D), jnp.float32),
        scratch_types=[
            pltpu.VMEM((JW, D), jnp.float32),
            pltpu.VMEM((NB, KI, JW, D), jnp.float32),
            pltpu.VMEM((NB, KI, JW, D), jnp.float32),
            pltpu.SemaphoreType.DMA((NB,)),
            pltpu.SemaphoreType.DMA((NB,)),
        ],
    )(_sc_body)
    return run(x, pe_weight)


# speedup vs baseline: 1.1834x; 1.0098x over previous
import functools

import jax
import jax.numpy as jnp
from jax import lax
from jax.experimental import pallas as pl
from jax.experimental.pallas import tpu as pltpu
from jax.experimental.pallas import tpu_sc as plsc

N = 512
D = 256
NW = 32
JW = N // NW
KI = 2
NB = 4
NCH = N // KI
LANES = 16


def _sc_body(x_hbm, pe_hbm, out_hbm, pe_v, bin_, bout, lsem, ssem):
    wid = lax.axis_index("s") * 2 + lax.axis_index("c")
    jb = wid * JW
    def x_slc(c):
        return x_hbm.at[pl.ds(c * KI, KI), pl.ds(jb, JW), :]

    def o_slc(c):
        return out_hbm.at[pl.ds(c * KI, KI), pl.ds(jb, JW), :]

    for b in range(NB):
        pltpu.async_copy(x_slc(b), bin_.at[b], lsem.at[b])

    pltpu.sync_copy(pe_hbm.at[pl.ds(jb, JW), :], pe_v)

    @pl.loop(0, NCH, step=NB)
    def _chunk_loop(c0):
        for b in range(NB):
            c = c0 + b
            pltpu.make_async_copy(x_slc(c), bin_.at[b], lsem.at[b]).wait()

            @pl.when(c >= NB)
            def _drain_out():
                pltpu.make_async_copy(
                    bout.at[b], o_slc(c - NB), ssem.at[b]
                ).wait()

            @plsc.parallel_loop(0, JW, unroll=2)
            def _j_loop(j):
                for dv in range(D // LANES):
                    dslc = pl.ds(dv * LANES, LANES)
                    pe_reg = pe_v[j, dslc]
                    for k in range(KI):
                        bout[b, k, j, dslc] = bin_[b, k, j, dslc] + pe_reg

            pltpu.async_copy(bout.at[b], o_slc(c), ssem.at[b])

            @pl.when(c + NB < NCH)
            def _refill():
                pltpu.async_copy(x_slc(c + NB), bin_.at[b], lsem.at[b])

    for b in range(NB):
        pltpu.make_async_copy(
            bout.at[b], o_slc(NCH - NB + b), ssem.at[b]
        ).wait()


@jax.jit
def kernel(x, pe_weight):
    mesh = plsc.VectorSubcoreMesh(core_axis_name="c", subcore_axis_name="s")
    run = functools.partial(
        pl.kernel,
        mesh=mesh,
        out_type=jax.ShapeDtypeStruct((N, N, D), jnp.float32),
        scratch_types=[
            pltpu.VMEM((JW, D), jnp.float32),
            pltpu.VMEM((NB, KI, JW, D), jnp.float32),
            pltpu.VMEM((NB, KI, JW, D), jnp.float32),
            pltpu.SemaphoreType.DMA((NB,)),
            pltpu.SemaphoreType.DMA((NB,)),
        ],
    )(_sc_body)
    return run(x, pe_weight)
